# baseline (speedup 1.0000x reference)
import functools

import numpy as np

import jax
import jax.numpy as jnp
from jax.experimental import pallas as pl
from jax.experimental.pallas import tpu as pltpu

_PAD = 128


def _up_conv_kernel(x_ref, d_ref, w_ref, b_ref, o_ref, se_ref, so_ref, *, H, W):
    Cin = x_ref.shape[1]
    W2 = 2 * W
    N2 = H * W2
    N4 = 2 * N2
    Cout = w_ref.shape[1]

    xb = x_ref[0].astype(jnp.bfloat16)
    xcol = jnp.dot(xb, d_ref[...],
                   preferred_element_type=jnp.float32).astype(jnp.bfloat16)

    z1 = jnp.zeros((Cin, 1), jnp.bfloat16)
    v = jnp.remainder(jax.lax.broadcasted_iota(jnp.int32, (1, N2), 1), W2)
    m = {
        0: jnp.where(v != 0, jnp.concatenate([z1, xcol[:, :N2 - 1]], axis=1),
                     jnp.bfloat16(0)),
        1: xcol,
        2: jnp.where(v != W2 - 1, jnp.concatenate([xcol[:, 1:], z1], axis=1),
                     jnp.bfloat16(0)),
    }

    z64 = jnp.zeros((Cin, W2), jnp.bfloat16)
    for dx in range(3):
        r0 = dx * Cin
        chunks = [m[dx][:, W2 * p:W2 * (p + 1)] for p in range(H)]
        so_ref[r0:r0 + Cin, 0:W2] = z64
        so_ref[r0:r0 + Cin, W2:2 * W2] = chunks[0]
        for p in range(H):
            nxt = chunks[p + 1] if p + 1 < H else z64
            se_ref[r0:r0 + Cin, _PAD + 2 * W2 * p:_PAD + 2 * W2 * (p + 1)] = (
                jnp.concatenate([chunks[p], chunks[p]], axis=1))
            so_ref[r0:r0 + Cin, _PAD + 2 * W2 * p:_PAD + 2 * W2 * (p + 1)] = (
                jnp.concatenate([chunks[p], nxt], axis=1))

    ones = jnp.ones((1, N4), jnp.bfloat16)
    acc = jnp.dot(b_ref[:, 0:1], ones, preferred_element_type=jnp.float32)
    acc = acc + jnp.dot(w_ref[1], se_ref[:, _PAD:_PAD + N4],
                        preferred_element_type=jnp.float32)
    acc = acc + jnp.dot(w_ref[2], so_ref[:, _PAD:_PAD + N4],
                        preferred_element_type=jnp.float32)
    acc = acc + jnp.dot(w_ref[0], so_ref[:, 0:N4],
                        preferred_element_type=jnp.float32)

    for y in range(2 * H):
        o_ref[0, :, y, :] = acc[:, W2 * y:W2 * (y + 1)]


@functools.lru_cache(maxsize=None)
def _dilation_matrix(H, W):
    d0 = np.repeat(np.eye(W, dtype=np.float32), 2, axis=1)
    return np.kron(np.eye(H, dtype=np.float32), d0)


def kernel(x_nchw, w_oihw, bias):
    B, Cin, H, W = x_nchw.shape
    Cout = w_oihw.shape[0]
    N = H * W

    x3 = x_nchw.reshape(B, Cin, N)
    dd = jnp.asarray(_dilation_matrix(H, W), dtype=jnp.bfloat16)
    wk = jnp.transpose(w_oihw, (2, 0, 3, 1)).reshape(3, Cout, 3 * Cin)
    wk = wk.astype(jnp.bfloat16)
    bb = jnp.broadcast_to(bias.astype(jnp.bfloat16)[:, None], (Cout, 128))

    flops = 2 * B * (N * 2 * N + 3 * 3 * Cin * Cout * 4 * N)
    bytes_accessed = int(x3.size * 4 + B * Cout * 4 * N * 4 + wk.size * 2)

    return pl.pallas_call(
        functools.partial(_up_conv_kernel, H=H, W=W),
        out_shape=jax.ShapeDtypeStruct((B, Cout, 2 * H, 2 * W), jnp.float32),
        grid=(B,),
        in_specs=[
            pl.BlockSpec((1, Cin, N), lambda i: (i, 0, 0)),
            pl.BlockSpec((N, 2 * N), lambda i: (0, 0)),
            pl.BlockSpec((3, Cout, 3 * Cin), lambda i: (0, 0, 0)),
            pl.BlockSpec((Cout, 128), lambda i: (0, 0)),
        ],
        out_specs=pl.BlockSpec((1, Cout, 2 * H, 2 * W), lambda i: (i, 0, 0, 0)),
        scratch_shapes=[
            pltpu.VMEM((3 * Cin, _PAD + 4 * N + _PAD), jnp.bfloat16),
            pltpu.VMEM((3 * Cin, _PAD + 4 * N + _PAD), jnp.bfloat16),
        ],
        compiler_params=pltpu.CompilerParams(
            dimension_semantics=("parallel",),
            vmem_limit_bytes=56 * 1024 * 1024),
        cost_estimate=pl.CostEstimate(
            flops=flops, transcendentals=0, bytes_accessed=bytes_accessed),
    )(x3, dd, wk, bb)


# speedup vs baseline: 2.8830x; 2.8830x over previous
import functools

import jax
import jax.numpy as jnp
from jax.experimental import pallas as pl
from jax.experimental.pallas import tpu as pltpu


def _fold_weights(w_oihw):
    w = jnp.transpose(w_oihw, (2, 3, 1, 0)).astype(jnp.float32)
    fold = {0: ((0,), (1, 2)), 1: ((0, 1), (2,))}
    phases = []
    for a in range(2):
        for c in range(2):
            for ky in range(2):
                for kx in range(2):
                    t = 0.0
                    for dy in fold[a][ky]:
                        for dx in fold[c][kx]:
                            t = t + w[dy, dx]
                    phases.append(t)
    cin, cout = phases[0].shape
    return jnp.stack(phases).reshape(2, 2, 4, cin, cout)


def _up_conv_kernel(x_ref, w_ref, b_ref, o_ref, xp_ref):
    bt, H, W, Cin = x_ref.shape
    Cout = w_ref.shape[4]
    M = bt * H * W

    xp_ref[:, 0:1, :, :] = jnp.zeros((bt, 1, W + 2, Cin), jnp.bfloat16)
    xp_ref[:, H + 1:H + 2, :, :] = jnp.zeros((bt, 1, W + 2, Cin), jnp.bfloat16)
    xp_ref[:, 1:H + 1, 0:1, :] = jnp.zeros((bt, H, 1, Cin), jnp.bfloat16)
    xp_ref[:, 1:H + 1, W + 1:W + 2, :] = jnp.zeros((bt, H, 1, Cin), jnp.bfloat16)
    xp_ref[:, 1:H + 1, 1:W + 1, :] = x_ref[...]

    bias2 = b_ref[...]
    for a in range(2):
        cols = []
        for c in range(2):
            acc = jnp.zeros((M, Cout), jnp.float32)
            for ky in range(2):
                for kx in range(2):
                    win = xp_ref[:, a + ky:a + ky + H, c + kx:c + kx + W, :]
                    acc = acc + jnp.dot(
                        win.reshape(M, Cin), w_ref[a, c, 2 * ky + kx],
                        preferred_element_type=jnp.float32)
            cols.append(acc)
        row = jnp.concatenate(cols, axis=-1) + bias2
        o_ref[:, :, a, :, :] = row.reshape(bt, H, W, 2 * Cout).astype(
            jnp.bfloat16)


def kernel(x_nchw, w_oihw, bias):
    B, Cin, H, W = x_nchw.shape
    Cout = w_oihw.shape[0]
    bt = next(d for d in (8, 4, 2, 1) if B % d == 0)

    xh = jnp.transpose(x_nchw, (0, 2, 3, 1)).astype(jnp.bfloat16)
    wf = _fold_weights(w_oihw).astype(jnp.bfloat16)
    b2 = jnp.concatenate([bias, bias]).reshape(1, 2 * Cout).astype(jnp.float32)

    flops = 2 * B * 4 * H * W * 4 * Cin * Cout
    bytes_accessed = int(xh.size * 2 + B * H * W * 4 * Cout * 2 + wf.size * 2)

    out = pl.pallas_call(
        _up_conv_kernel,
        out_shape=jax.ShapeDtypeStruct((B, H, 2, W, 2 * Cout), jnp.bfloat16),
        grid=(B // bt,),
        in_specs=[
            pl.BlockSpec((bt, H, W, Cin), lambda i: (i, 0, 0, 0)),
            pl.BlockSpec((2, 2, 4, Cin, Cout), lambda i: (0, 0, 0, 0, 0)),
            pl.BlockSpec((1, 2 * Cout), lambda i: (0, 0)),
        ],
        out_specs=pl.BlockSpec((bt, H, 2, W, 2 * Cout),
                               lambda i: (i, 0, 0, 0, 0)),
        scratch_shapes=[pltpu.VMEM((bt, H + 2, W + 2, Cin), jnp.bfloat16)],
        compiler_params=pltpu.CompilerParams(
            dimension_semantics=("parallel",),
            vmem_limit_bytes=56 * 1024 * 1024),
        cost_estimate=pl.CostEstimate(
            flops=flops, transcendentals=0, bytes_accessed=bytes_accessed),
    )(xh, wf, b2)

    y = out.reshape(B, H, 2, W, 2, Cout).transpose(0, 5, 1, 2, 3, 4)
    return y.astype(jnp.float32).reshape(B, Cout, 2 * H, 2 * W)


# speedup vs baseline: 3.2079x; 1.1127x over previous
import functools

import jax
import jax.numpy as jnp
from jax.experimental import pallas as pl
from jax.experimental.pallas import tpu as pltpu


def _fold_weights(w_oihw):
    w = jnp.transpose(w_oihw, (2, 3, 1, 0)).astype(jnp.float32)
    fold = {0: ((0,), (1, 2)), 1: ((0, 1), (2,))}
    phases = []
    for a in range(2):
        for c in range(2):
            for ky in range(2):
                for kx in range(2):
                    t = 0.0
                    for dy in fold[a][ky]:
                        for dx in fold[c][kx]:
                            t = t + w[dy, dx]
                    phases.append(t)
    cin, cout = phases[0].shape
    return jnp.stack(phases).reshape(2, 2, 4, cin, cout)


def _up_conv_kernel(x_ref, w_ref, b_ref, o_ref, xp_ref):
    bt, H, W, Cin = x_ref.shape
    Cout = w_ref.shape[4]
    M = bt * H * W

    xp_ref[:, 0:1, :, :] = jnp.zeros((bt, 1, W + 2, Cin), jnp.bfloat16)
    xp_ref[:, H + 1:H + 2, :, :] = jnp.zeros((bt, 1, W + 2, Cin), jnp.bfloat16)
    xp_ref[:, 1:H + 1, 0:1, :] = jnp.zeros((bt, H, 1, Cin), jnp.bfloat16)
    xp_ref[:, 1:H + 1, W + 1:W + 2, :] = jnp.zeros((bt, H, 1, Cin), jnp.bfloat16)
    xp_ref[:, 1:H + 1, 1:W + 1, :] = x_ref[...]

    bias2 = b_ref[...]
    win = {}
    for dy in range(3):
        for dx in range(3):
            win[dy, dx] = xp_ref[:, dy:dy + H, dx:dx + W, :].reshape(M, Cin)
    for a in range(2):
        cols = []
        for c in range(2):
            lhs = jnp.concatenate(
                [win[a + ky, c + kx] for ky in range(2) for kx in range(2)],
                axis=1)
            acc = jnp.dot(lhs, w_ref[a, c].reshape(4 * Cin, Cout),
                          preferred_element_type=jnp.float32)
            cols.append(acc)
        row = jnp.concatenate(cols, axis=-1) + bias2
        o_ref[:, :, a, :, :] = row.reshape(bt, H, W, 2 * Cout).astype(
            jnp.bfloat16)


def kernel(x_nchw, w_oihw, bias):
    B, Cin, H, W = x_nchw.shape
    Cout = w_oihw.shape[0]
    bt = next(d for d in (8, 4, 2, 1) if B % d == 0)

    xh = jnp.transpose(x_nchw, (0, 2, 3, 1)).astype(jnp.bfloat16)
    wf = _fold_weights(w_oihw).astype(jnp.bfloat16)
    b2 = jnp.concatenate([bias, bias]).reshape(1, 2 * Cout).astype(jnp.float32)

    flops = 2 * B * 4 * H * W * 4 * Cin * Cout
    bytes_accessed = int(xh.size * 2 + B * H * W * 4 * Cout * 2 + wf.size * 2)

    out = pl.pallas_call(
        _up_conv_kernel,
        out_shape=jax.ShapeDtypeStruct((B, H, 2, W, 2 * Cout), jnp.bfloat16),
        grid=(B // bt,),
        in_specs=[
            pl.BlockSpec((bt, H, W, Cin), lambda i: (i, 0, 0, 0)),
            pl.BlockSpec((2, 2, 4, Cin, Cout), lambda i: (0, 0, 0, 0, 0)),
            pl.BlockSpec((1, 2 * Cout), lambda i: (0, 0)),
        ],
        out_specs=pl.BlockSpec((bt, H, 2, W, 2 * Cout),
                               lambda i: (i, 0, 0, 0, 0)),
        scratch_shapes=[pltpu.VMEM((bt, H + 2, W + 2, Cin), jnp.bfloat16)],
        compiler_params=pltpu.CompilerParams(
            dimension_semantics=("parallel",),
            vmem_limit_bytes=56 * 1024 * 1024),
        cost_estimate=pl.CostEstimate(
            flops=flops, transcendentals=0, bytes_accessed=bytes_accessed),
    )(xh, wf, b2)

    y = out.reshape(B, H, 2, W, 2, Cout).transpose(0, 5, 1, 2, 3, 4)
    return y.astype(jnp.float32).reshape(B, Cout, 2 * H, 2 * W)
